# Initial kernel scaffold; baseline (speedup 1.0000x reference)
import functools

import jax
import jax.numpy as jnp
from jax import lax
from jax.experimental import pallas as pl
from jax.experimental.pallas import tpu as pltpu
from jax.experimental.pallas import tpu_sc as plsc

NC = 2
NS = 16
NW = NC * NS
CHUNK = 128
BLK = 1280


def _mesh():
    return plsc.VectorSubcoreMesh(
        core_axis_name="c", subcore_axis_name="s", num_cores=NC, num_subcores=NS
    )


def _make_hist(n_pad, e_pad):
    epw = e_pad // NW
    nchunks = epw // CHUNK
    rpt = n_pad // NS

    @functools.partial(
        pl.kernel,
        out_type=(
            jax.ShapeDtypeStruct((NC, n_pad), jnp.float32),
            jax.ShapeDtypeStruct((NC, n_pad), jnp.float32),
        ),
        mesh=_mesh(),
        scratch_types=[
            pltpu.VMEM((CHUNK,), jnp.int32),
            pltpu.VMEM((CHUNK,), jnp.int32),
            pltpu.VMEM((CHUNK,), jnp.float32),
            pltpu.VMEM_SHARED((n_pad,), jnp.float32),
            pltpu.VMEM_SHARED((n_pad,), jnp.float32),
        ],
    )
    def hist(src_hbm, dst_hbm, ones_hbm, z1_hbm, outs_hbm, outd_hbm,
             idx_s, idx_d, ones_v, acc_s, acc_d):
        c = lax.axis_index("c")
        s = lax.axis_index("s")
        wid = s * NC + c
        base = wid * epw
        r0 = s * rpt
        pltpu.sync_copy(ones_hbm, ones_v)
        pltpu.sync_copy(z1_hbm, acc_s.at[pl.ds(r0, rpt)])
        pltpu.sync_copy(z1_hbm, acc_d.at[pl.ds(r0, rpt)])
        plsc.subcore_barrier()

        def body(i, carry):
            off = base + i * CHUNK
            pltpu.sync_copy(src_hbm.at[pl.ds(off, CHUNK)], idx_s)
            pltpu.sync_copy(dst_hbm.at[pl.ds(off, CHUNK)], idx_d)
            pltpu.sync_copy(ones_v, acc_s.at[idx_s], add=True)
            pltpu.sync_copy(ones_v, acc_d.at[idx_d], add=True)
            return carry

        lax.fori_loop(0, nchunks, body, 0)
        plsc.subcore_barrier()
        pltpu.sync_copy(acc_s.at[pl.ds(r0, rpt)], outs_hbm.at[c, pl.ds(r0, rpt)])
        pltpu.sync_copy(acc_d.at[pl.ds(r0, rpt)], outd_hbm.at[c, pl.ds(r0, rpt)])

    return hist


def _make_segsum(n_pad, d, e_pad):
    epw = e_pad // NW
    nchunks = epw // CHUNK
    rpt = n_pad // NS

    @functools.partial(
        pl.kernel,
        out_type=jax.ShapeDtypeStruct((NC, n_pad, d), jnp.float32),
        mesh=_mesh(),
        scratch_types=[
            pltpu.VMEM((CHUNK,), jnp.int32),
            pltpu.VMEM((CHUNK,), jnp.int32),
            pltpu.VMEM((CHUNK, d), jnp.float32),
            pltpu.VMEM_SHARED((n_pad, d), jnp.float32),
            pltpu.SemaphoreType.DMA,
        ],
    )
    def segsum(y_hbm, src_hbm, dst_hbm, zrow_hbm, out_hbm,
               idx_s, idx_d, rows, acc, sem):
        c = lax.axis_index("c")
        s = lax.axis_index("s")
        wid = s * NC + c
        base = wid * epw
        r0 = s * rpt
        pltpu.sync_copy(zrow_hbm, acc.at[pl.ds(r0, rpt)])
        plsc.subcore_barrier()

        def body(i, carry):
            off = base + i * CHUNK
            pltpu.sync_copy(src_hbm.at[pl.ds(off, CHUNK)], idx_s)
            pltpu.sync_copy(dst_hbm.at[pl.ds(off, CHUNK)], idx_d)
            pltpu.async_copy(y_hbm.at[idx_s], rows, sem).wait()
            pltpu.sync_copy(rows, acc.at[idx_d], add=True)
            return carry

        lax.fori_loop(0, nchunks, body, 0)
        plsc.subcore_barrier()
        pltpu.sync_copy(acc.at[pl.ds(r0, rpt)], out_hbm.at[c, pl.ds(r0, rpt)])

    return segsum


def _scale_body(hs_ref, hd_ref, x_ref, y_ref, dinv_ref, dinv2_ref):
    deg = hs_ref[0, :] + hs_ref[1, :]
    deg2 = hd_ref[0, :] + hd_ref[1, :] + 1.0
    dinv = jnp.where(deg > 0.0, lax.rsqrt(deg), 0.0)
    dinv2 = lax.rsqrt(deg2)
    dinv_ref[...] = dinv[:, None]
    dinv2_ref[...] = dinv2[:, None]
    y_ref[...] = x_ref[...] * dinv[:, None]


def _gates_body(x_ref, s_ref, dinv_ref, dinv2_ref, wz_ref, wh_ref, wg_ref,
                bz_ref, bh_ref, y2_ref):
    tx1 = (s_ref[0] + s_ref[1]) * (-dinv_ref[...])
    xb = x_ref[...]
    az = (jnp.dot(xb, wz_ref[0], preferred_element_type=jnp.float32)
          + jnp.dot(tx1, wz_ref[1], preferred_element_type=jnp.float32)
          + bz_ref[...])
    ah = (jnp.dot(xb, wh_ref[0], preferred_element_type=jnp.float32)
          + jnp.dot(tx1, wh_ref[1], preferred_element_type=jnp.float32)
          + bh_ref[...])
    hn = (1.0 - jax.nn.sigmoid(az)) * jnp.tanh(ah)
    y2_ref[...] = jnp.dot(hn, wg_ref[...],
                          preferred_element_type=jnp.float32) * dinv2_ref[...]


def _final_body(s2_ref, y2_ref, dinv2_ref, bg_ref, wl_ref, bl_ref, out_ref):
    t = (s2_ref[0] + s2_ref[1] + y2_ref[...]) * dinv2_ref[...] + bg_ref[...]
    h1 = jnp.maximum(t, 0.0)
    out_ref[...] = jnp.sum(h1 * wl_ref[...], axis=1) + bl_ref[0]


def kernel(x, edge_index, Wx_z, bx_z, Wh_z, bh_z, Wx_r, bx_r, Wh_r, bh_r,
           Wx_h, bx_h, Wh_h, bh_h, W_gcn, b_gcn, W_lin, b_lin):
    n, d = x.shape
    e = edge_index.shape[1]
    n_pad = -(-n // BLK) * BLK
    e_pad = -(-e // (NW * CHUNK)) * (NW * CHUNK)
    grid = n_pad // BLK
    trash = n_pad - n

    pad_idx = n + (jnp.arange(e_pad - e, dtype=jnp.int32) % trash)
    srcp = jnp.concatenate([edge_index[0], pad_idx])
    dstp = jnp.concatenate([edge_index[1], pad_idx])
    xp = jnp.concatenate([x, jnp.zeros((trash, d), x.dtype)], axis=0)
    zrow = jnp.zeros((n_pad // NS, d), jnp.float32)
    z1 = jnp.zeros((n_pad // NS,), jnp.float32)
    ones_c = jnp.ones((CHUNK,), jnp.float32)

    hs, hd = _make_hist(n_pad, e_pad)(srcp, dstp, ones_c, z1)

    y, dinv, dinv2 = pl.pallas_call(
        _scale_body,
        grid=(grid,),
        in_specs=[
            pl.BlockSpec((NC, BLK), lambda i: (0, i)),
            pl.BlockSpec((NC, BLK), lambda i: (0, i)),
            pl.BlockSpec((BLK, d), lambda i: (i, 0)),
        ],
        out_specs=[
            pl.BlockSpec((BLK, d), lambda i: (i, 0)),
            pl.BlockSpec((BLK, 1), lambda i: (i, 0)),
            pl.BlockSpec((BLK, 1), lambda i: (i, 0)),
        ],
        out_shape=[
            jax.ShapeDtypeStruct((n_pad, d), jnp.float32),
            jax.ShapeDtypeStruct((n_pad, 1), jnp.float32),
            jax.ShapeDtypeStruct((n_pad, 1), jnp.float32),
        ],
    )(hs, hd, xp)

    segsum = _make_segsum(n_pad, d, e_pad)

    s_part = segsum(y, srcp, dstp, zrow)

    bz = bx_z + bh_z
    bh = bx_h + bh_h
    y2 = pl.pallas_call(
        _gates_body,
        grid=(grid,),
        in_specs=[
            pl.BlockSpec((BLK, d), lambda i: (i, 0)),
            pl.BlockSpec((NC, BLK, d), lambda i: (0, i, 0)),
            pl.BlockSpec((BLK, 1), lambda i: (i, 0)),
            pl.BlockSpec((BLK, 1), lambda i: (i, 0)),
            pl.BlockSpec(Wx_z.shape, lambda i: (0, 0, 0)),
            pl.BlockSpec(Wx_h.shape, lambda i: (0, 0, 0)),
            pl.BlockSpec(W_gcn.shape, lambda i: (0, 0)),
            pl.BlockSpec(bz.shape, lambda i: (0,)),
            pl.BlockSpec(bh.shape, lambda i: (0,)),
        ],
        out_specs=pl.BlockSpec((BLK, d), lambda i: (i, 0)),
        out_shape=jax.ShapeDtypeStruct((n_pad, d), jnp.float32),
    )(xp, s_part, dinv, dinv2, Wx_z, Wx_h, W_gcn, bz, bh)

    s2_part = segsum(y2, srcp, dstp, zrow)

    wl_row = W_lin.reshape(1, -1)
    outp = pl.pallas_call(
        _final_body,
        grid=(grid,),
        in_specs=[
            pl.BlockSpec((NC, BLK, d), lambda i: (0, i, 0)),
            pl.BlockSpec((BLK, d), lambda i: (i, 0)),
            pl.BlockSpec((BLK, 1), lambda i: (i, 0)),
            pl.BlockSpec(b_gcn.shape, lambda i: (0,)),
            pl.BlockSpec((1, d), lambda i: (0, 0)),
            pl.BlockSpec(b_lin.shape, lambda i: (0,)),
        ],
        out_specs=pl.BlockSpec((BLK,), lambda i: (i,)),
        out_shape=jax.ShapeDtypeStruct((n_pad,), jnp.float32),
    )(s2_part, y2, dinv2, b_gcn, wl_row, b_lin)

    return outp[:n]


# speedup vs baseline: 25.2284x; 25.2284x over previous
import functools

import jax
import jax.numpy as jnp
from jax import lax
from jax.experimental import pallas as pl
from jax.experimental.pallas import tpu as pltpu
from jax.experimental.pallas import tpu_sc as plsc

NC = 2
NS = 16
NW = NC * NS
CHUNK = 128
BLK = 1280


def _mesh():
    return plsc.VectorSubcoreMesh(
        core_axis_name="c", subcore_axis_name="s", num_cores=NC, num_subcores=NS
    )


def _make_hist(n_pad, e_pad):
    epw = e_pad // NW
    nchunks = epw // CHUNK
    rpt = n_pad // NS

    @functools.partial(
        pl.kernel,
        out_type=(
            jax.ShapeDtypeStruct((NC, n_pad), jnp.float32),
            jax.ShapeDtypeStruct((NC, n_pad), jnp.float32),
        ),
        mesh=_mesh(),
        scratch_types=[
            pltpu.VMEM((CHUNK,), jnp.int32),
            pltpu.VMEM((CHUNK,), jnp.int32),
            pltpu.VMEM((CHUNK,), jnp.float32),
            pltpu.VMEM_SHARED((n_pad,), jnp.float32),
            pltpu.VMEM_SHARED((n_pad,), jnp.float32),
        ],
    )
    def hist(src_hbm, dst_hbm, ones_hbm, z1_hbm, outs_hbm, outd_hbm,
             idx_s, idx_d, ones_v, acc_s, acc_d):
        c = lax.axis_index("c")
        s = lax.axis_index("s")
        wid = s * NC + c
        base = wid * epw
        r0 = s * rpt
        pltpu.sync_copy(ones_hbm, ones_v)
        pltpu.sync_copy(z1_hbm, acc_s.at[pl.ds(r0, rpt)])
        pltpu.sync_copy(z1_hbm, acc_d.at[pl.ds(r0, rpt)])
        plsc.subcore_barrier()

        def body(i, carry):
            off = base + i * CHUNK
            pltpu.sync_copy(src_hbm.at[pl.ds(off, CHUNK)], idx_s)
            pltpu.sync_copy(dst_hbm.at[pl.ds(off, CHUNK)], idx_d)
            pltpu.sync_copy(ones_v, acc_s.at[idx_s], add=True)
            pltpu.sync_copy(ones_v, acc_d.at[idx_d], add=True)
            return carry

        lax.fori_loop(0, nchunks, body, 0)
        plsc.subcore_barrier()
        pltpu.sync_copy(acc_s.at[pl.ds(r0, rpt)], outs_hbm.at[c, pl.ds(r0, rpt)])
        pltpu.sync_copy(acc_d.at[pl.ds(r0, rpt)], outd_hbm.at[c, pl.ds(r0, rpt)])

    return hist


def _make_segsum(n_pad, d, e_pad):
    epw = e_pad // NW
    nchunks = epw // CHUNK
    rpt = n_pad // NS

    @functools.partial(
        pl.kernel,
        out_type=jax.ShapeDtypeStruct((NC, n_pad, d), jnp.float32),
        mesh=_mesh(),
        scratch_types=[
            pltpu.VMEM((CHUNK,), jnp.int32),
            pltpu.VMEM((CHUNK,), jnp.int32),
            pltpu.VMEM((CHUNK, d), jnp.float32),
            pltpu.VMEM_SHARED((n_pad, d), jnp.float32),
            pltpu.SemaphoreType.DMA,
        ],
    )
    def segsum(y_hbm, src_hbm, dst_hbm, zrow_hbm, out_hbm,
               idx_s, idx_d, rows, acc, sem):
        c = lax.axis_index("c")
        s = lax.axis_index("s")
        wid = s * NC + c
        base = wid * epw
        r0 = s * rpt
        pltpu.sync_copy(zrow_hbm, acc.at[pl.ds(r0, rpt)])
        plsc.subcore_barrier()

        def body(i, carry):
            off = base + i * CHUNK
            pltpu.sync_copy(src_hbm.at[pl.ds(off, CHUNK)], idx_s)
            pltpu.sync_copy(dst_hbm.at[pl.ds(off, CHUNK)], idx_d)
            pltpu.async_copy(y_hbm.at[idx_s], rows, sem).wait()
            pltpu.sync_copy(rows, acc.at[idx_d], add=True)
            return carry

        lax.fori_loop(0, nchunks, body, 0)
        plsc.subcore_barrier()
        pltpu.sync_copy(acc.at[pl.ds(r0, rpt)], out_hbm.at[c, pl.ds(r0, rpt)])

    return segsum


def _scale_body(hs_ref, hd_ref, x_ref, y_ref, dinv_ref, dinv2_ref):
    deg = hs_ref[0, :] + hs_ref[1, :]
    deg2 = hd_ref[0, :] + hd_ref[1, :] + 1.0
    dinv = jnp.where(deg > 0.0, lax.rsqrt(deg), 0.0)
    dinv2 = lax.rsqrt(deg2)
    dinv_ref[...] = dinv[:, None]
    dinv2_ref[...] = dinv2[:, None]
    y_ref[...] = x_ref[...] * dinv[:, None]


def _gates_body(x_ref, s_ref, dinv_ref, dinv2_ref, wz_ref, wh_ref, wg_ref,
                bz_ref, bh_ref, y2_ref):
    tx1 = (s_ref[0] + s_ref[1]) * (-dinv_ref[...])
    xb = x_ref[...]
    az = (jnp.dot(xb, wz_ref[0], preferred_element_type=jnp.float32)
          + jnp.dot(tx1, wz_ref[1], preferred_element_type=jnp.float32)
          + bz_ref[...])
    ah = (jnp.dot(xb, wh_ref[0], preferred_element_type=jnp.float32)
          + jnp.dot(tx1, wh_ref[1], preferred_element_type=jnp.float32)
          + bh_ref[...])
    hn = (1.0 - jax.nn.sigmoid(az)) * jnp.tanh(ah)
    y2_ref[...] = jnp.dot(hn, wg_ref[...],
                          preferred_element_type=jnp.float32) * dinv2_ref[...]


def _final_body(s2_ref, y2_ref, dinv2_ref, bg_ref, wl_ref, bl_ref, out_ref):
    t = (s2_ref[0] + s2_ref[1] + y2_ref[...]) * dinv2_ref[...] + bg_ref[...]
    h1 = jnp.maximum(t, 0.0)
    out_ref[...] = (jnp.sum(h1 * wl_ref[...], axis=1) + bl_ref[0])[:, None]


def kernel(x, edge_index, Wx_z, bx_z, Wh_z, bh_z, Wx_r, bx_r, Wh_r, bh_r,
           Wx_h, bx_h, Wh_h, bh_h, W_gcn, b_gcn, W_lin, b_lin):
    n, d = x.shape
    e = edge_index.shape[1]
    n_pad = -(-n // BLK) * BLK
    e_pad = -(-e // (NW * CHUNK)) * (NW * CHUNK)
    grid = n_pad // BLK
    trash = n_pad - n

    pad_idx = n + (jnp.arange(e_pad - e, dtype=jnp.int32) % trash)
    srcp = jnp.concatenate([edge_index[0], pad_idx])
    dstp = jnp.concatenate([edge_index[1], pad_idx])
    xp = jnp.concatenate([x, jnp.zeros((trash, d), x.dtype)], axis=0)
    zrow = jnp.zeros((n_pad // NS, d), jnp.float32)
    z1 = jnp.zeros((n_pad // NS,), jnp.float32)
    ones_c = jnp.ones((CHUNK,), jnp.float32)

    hs, hd = _make_hist(n_pad, e_pad)(srcp, dstp, ones_c, z1)

    y, dinv, dinv2 = pl.pallas_call(
        _scale_body,
        grid=(grid,),
        in_specs=[
            pl.BlockSpec((NC, BLK), lambda i: (0, i)),
            pl.BlockSpec((NC, BLK), lambda i: (0, i)),
            pl.BlockSpec((BLK, d), lambda i: (i, 0)),
        ],
        out_specs=[
            pl.BlockSpec((BLK, d), lambda i: (i, 0)),
            pl.BlockSpec((BLK, 1), lambda i: (i, 0)),
            pl.BlockSpec((BLK, 1), lambda i: (i, 0)),
        ],
        out_shape=[
            jax.ShapeDtypeStruct((n_pad, d), jnp.float32),
            jax.ShapeDtypeStruct((n_pad, 1), jnp.float32),
            jax.ShapeDtypeStruct((n_pad, 1), jnp.float32),
        ],
    )(hs, hd, xp)

    segsum = _make_segsum(n_pad, d, e_pad)

    s_part = segsum(y, srcp, dstp, zrow)

    bz = bx_z + bh_z
    bh = bx_h + bh_h
    y2 = pl.pallas_call(
        _gates_body,
        grid=(grid,),
        in_specs=[
            pl.BlockSpec((BLK, d), lambda i: (i, 0)),
            pl.BlockSpec((NC, BLK, d), lambda i: (0, i, 0)),
            pl.BlockSpec((BLK, 1), lambda i: (i, 0)),
            pl.BlockSpec((BLK, 1), lambda i: (i, 0)),
            pl.BlockSpec(Wx_z.shape, lambda i: (0, 0, 0)),
            pl.BlockSpec(Wx_h.shape, lambda i: (0, 0, 0)),
            pl.BlockSpec(W_gcn.shape, lambda i: (0, 0)),
            pl.BlockSpec(bz.shape, lambda i: (0,)),
            pl.BlockSpec(bh.shape, lambda i: (0,)),
        ],
        out_specs=pl.BlockSpec((BLK, d), lambda i: (i, 0)),
        out_shape=jax.ShapeDtypeStruct((n_pad, d), jnp.float32),
    )(xp, s_part, dinv, dinv2, Wx_z, Wx_h, W_gcn, bz, bh)

    s2_part = segsum(y2, srcp, dstp, zrow)

    wl_row = W_lin.reshape(1, -1)
    outp = pl.pallas_call(
        _final_body,
        grid=(grid,),
        in_specs=[
            pl.BlockSpec((NC, BLK, d), lambda i: (0, i, 0)),
            pl.BlockSpec((BLK, d), lambda i: (i, 0)),
            pl.BlockSpec((BLK, 1), lambda i: (i, 0)),
            pl.BlockSpec(b_gcn.shape, lambda i: (0,)),
            pl.BlockSpec((1, d), lambda i: (0, 0)),
            pl.BlockSpec(b_lin.shape, lambda i: (0,)),
        ],
        out_specs=pl.BlockSpec((BLK, 1), lambda i: (i, 0)),
        out_shape=jax.ShapeDtypeStruct((n_pad, 1), jnp.float32),
    )(s2_part, y2, dinv2, b_gcn, wl_row, b_lin)

    return outp[:n, 0]


# speedup vs baseline: 45.3050x; 1.7958x over previous
import functools

import jax
import jax.numpy as jnp
from jax import lax
from jax.experimental import pallas as pl
from jax.experimental.pallas import tpu as pltpu
from jax.experimental.pallas import tpu_sc as plsc

NC = 2
NS = 16
NW = NC * NS
CHUNK = 128
BLK = 1280


def _mesh():
    return plsc.VectorSubcoreMesh(
        core_axis_name="c", subcore_axis_name="s", num_cores=NC, num_subcores=NS
    )


def _make_hist(n_pad, e_pad):
    epw = e_pad // NW
    nt = epw // CHUNK
    rpt = n_pad // NS

    @functools.partial(
        pl.kernel,
        out_type=(
            jax.ShapeDtypeStruct((NC, n_pad), jnp.float32),
            jax.ShapeDtypeStruct((NC, n_pad), jnp.float32),
        ),
        mesh=_mesh(),
        scratch_types=[
            pltpu.VMEM((nt, CHUNK), jnp.int32),
            pltpu.VMEM((nt, CHUNK), jnp.int32),
            pltpu.VMEM((CHUNK,), jnp.float32),
            pltpu.VMEM_SHARED((n_pad,), jnp.float32),
            pltpu.VMEM_SHARED((n_pad,), jnp.float32),
            pltpu.SemaphoreType.DMA((2,)),
        ],
    )
    def hist(src_hbm, dst_hbm, ones_hbm, z1_hbm, outs_hbm, outd_hbm,
             idx_s, idx_d, ones_v, acc_s, acc_d, sem):
        c = lax.axis_index("c")
        s = lax.axis_index("s")
        wid = s * NC + c
        r0 = s * rpt
        pltpu.sync_copy(ones_hbm, ones_v)
        pltpu.sync_copy(z1_hbm, acc_s.at[pl.ds(r0, rpt)])
        pltpu.sync_copy(z1_hbm, acc_d.at[pl.ds(r0, rpt)])
        pltpu.sync_copy(src_hbm.at[wid], idx_s)
        pltpu.sync_copy(dst_hbm.at[wid], idx_d)
        plsc.subcore_barrier()

        def body(i, carry):
            pltpu.async_copy(ones_v, acc_s.at[idx_s.at[i]], sem.at[0], add=True)
            pltpu.async_copy(ones_v, acc_d.at[idx_d.at[i]], sem.at[1], add=True)
            return carry

        lax.fori_loop(0, nt, body, 0)
        pltpu.make_async_copy(src_hbm.at[0], idx_s, sem.at[0]).wait()
        pltpu.make_async_copy(dst_hbm.at[0], idx_d, sem.at[1]).wait()
        plsc.subcore_barrier()
        pltpu.sync_copy(acc_s.at[pl.ds(r0, rpt)], outs_hbm.at[c, pl.ds(r0, rpt)])
        pltpu.sync_copy(acc_d.at[pl.ds(r0, rpt)], outd_hbm.at[c, pl.ds(r0, rpt)])

    return hist


def _make_segsum(n_pad, d, e_pad):
    epw = e_pad // NW
    nt = epw // CHUNK
    rpt = n_pad // NS
    NB = 2
    G = 16
    ngroups = nt // G

    @functools.partial(
        pl.kernel,
        out_type=jax.ShapeDtypeStruct((NC, n_pad, d), jnp.float32),
        mesh=_mesh(),
        scratch_types=[
            pltpu.VMEM((G, CHUNK), jnp.int32),
            pltpu.VMEM((G, CHUNK), jnp.int32),
            pltpu.VMEM((NB, CHUNK, d), jnp.float32),
            pltpu.VMEM_SHARED((n_pad, d), jnp.float32),
            pltpu.SemaphoreType.DMA((NB,)),
            pltpu.SemaphoreType.DMA((NB,)),
        ],
    )
    def segsum(y_hbm, src_hbm, dst_hbm, zrow_hbm, out_hbm,
               idx_s, idx_d, rows, acc, gsem, ssem):
        c = lax.axis_index("c")
        s = lax.axis_index("s")
        wid = s * NC + c
        r0 = s * rpt
        pltpu.sync_copy(zrow_hbm, acc.at[pl.ds(r0, rpt)])
        plsc.subcore_barrier()

        def group(g, carry):
            pltpu.sync_copy(src_hbm.at[wid, pl.ds(g * G, G)], idx_s)
            pltpu.sync_copy(dst_hbm.at[wid, pl.ds(g * G, G)], idx_d)
            pltpu.async_copy(y_hbm.at[idx_s.at[0]], rows.at[0], gsem.at[0])

            def body(j, carry2):
                b = lax.rem(j, NB)
                pltpu.make_async_copy(
                    y_hbm.at[pl.ds(0, CHUNK)], rows.at[b], gsem.at[b]).wait()
                pltpu.async_copy(rows.at[b], acc.at[idx_d.at[j]], ssem.at[b],
                                 add=True)

                @pl.when(j + 1 < G)
                def _():
                    b1 = lax.rem(j + 1, NB)

                    @pl.when(j >= NB - 1)
                    def _():
                        pltpu.make_async_copy(
                            y_hbm.at[pl.ds(0, CHUNK)], rows.at[b1],
                            ssem.at[b1]).wait()

                    pltpu.async_copy(y_hbm.at[idx_s.at[j + 1]], rows.at[b1],
                                     gsem.at[b1])

                return carry2

            lax.fori_loop(0, G, body, 0)
            for k in range(NB):
                bb = (G - NB + k) % NB
                pltpu.make_async_copy(
                    y_hbm.at[pl.ds(0, CHUNK)], rows.at[bb], ssem.at[bb]).wait()
            return carry

        lax.fori_loop(0, ngroups, group, 0)
        plsc.subcore_barrier()
        pltpu.sync_copy(acc.at[pl.ds(r0, rpt)], out_hbm.at[c, pl.ds(r0, rpt)])

    return segsum


def _scale_body(hs_ref, hd_ref, x_ref, y_ref, dinv_ref, dinv2_ref):
    deg = hs_ref[0, :] + hs_ref[1, :]
    deg2 = hd_ref[0, :] + hd_ref[1, :] + 1.0
    dinv = jnp.where(deg > 0.0, lax.rsqrt(deg), 0.0)
    dinv2 = lax.rsqrt(deg2)
    dinv_ref[...] = dinv[:, None]
    dinv2_ref[...] = dinv2[:, None]
    y_ref[...] = x_ref[...] * dinv[:, None]


def _gates_body(x_ref, s_ref, dinv_ref, dinv2_ref, wz_ref, wh_ref, wg_ref,
                bz_ref, bh_ref, y2_ref):
    tx1 = (s_ref[0] + s_ref[1]) * (-dinv_ref[...])
    xb = x_ref[...]
    az = (jnp.dot(xb, wz_ref[0], preferred_element_type=jnp.float32)
          + jnp.dot(tx1, wz_ref[1], preferred_element_type=jnp.float32)
          + bz_ref[...])
    ah = (jnp.dot(xb, wh_ref[0], preferred_element_type=jnp.float32)
          + jnp.dot(tx1, wh_ref[1], preferred_element_type=jnp.float32)
          + bh_ref[...])
    hn = (1.0 - jax.nn.sigmoid(az)) * jnp.tanh(ah)
    y2_ref[...] = jnp.dot(hn, wg_ref[...],
                          preferred_element_type=jnp.float32) * dinv2_ref[...]


def _final_body(s2_ref, y2_ref, dinv2_ref, bg_ref, wl_ref, bl_ref, out_ref):
    t = (s2_ref[0] + s2_ref[1] + y2_ref[...]) * dinv2_ref[...] + bg_ref[...]
    h1 = jnp.maximum(t, 0.0)
    out_ref[...] = (jnp.sum(h1 * wl_ref[...], axis=1) + bl_ref[0])[:, None]


def kernel(x, edge_index, Wx_z, bx_z, Wh_z, bh_z, Wx_r, bx_r, Wh_r, bh_r,
           Wx_h, bx_h, Wh_h, bh_h, W_gcn, b_gcn, W_lin, b_lin):
    n, d = x.shape
    e = edge_index.shape[1]
    n_pad = -(-n // BLK) * BLK
    e_pad = -(-e // (NW * CHUNK * 8)) * (NW * CHUNK * 8)
    grid = n_pad // BLK
    trash = n_pad - n

    pad_idx = n + (jnp.arange(e_pad - e, dtype=jnp.int32) % trash)
    srcp = jnp.concatenate([edge_index[0], pad_idx]).reshape(NW, -1, CHUNK)
    dstp = jnp.concatenate([edge_index[1], pad_idx]).reshape(NW, -1, CHUNK)
    xp = jnp.concatenate([x, jnp.zeros((trash, d), x.dtype)], axis=0)
    zrow = jnp.zeros((n_pad // NS, d), jnp.float32)
    z1 = jnp.zeros((n_pad // NS,), jnp.float32)
    ones_c = jnp.ones((CHUNK,), jnp.float32)

    hs, hd = _make_hist(n_pad, e_pad)(srcp, dstp, ones_c, z1)

    y, dinv, dinv2 = pl.pallas_call(
        _scale_body,
        grid=(grid,),
        in_specs=[
            pl.BlockSpec((NC, BLK), lambda i: (0, i)),
            pl.BlockSpec((NC, BLK), lambda i: (0, i)),
            pl.BlockSpec((BLK, d), lambda i: (i, 0)),
        ],
        out_specs=[
            pl.BlockSpec((BLK, d), lambda i: (i, 0)),
            pl.BlockSpec((BLK, 1), lambda i: (i, 0)),
            pl.BlockSpec((BLK, 1), lambda i: (i, 0)),
        ],
        out_shape=[
            jax.ShapeDtypeStruct((n_pad, d), jnp.float32),
            jax.ShapeDtypeStruct((n_pad, 1), jnp.float32),
            jax.ShapeDtypeStruct((n_pad, 1), jnp.float32),
        ],
    )(hs, hd, xp)

    segsum = _make_segsum(n_pad, d, e_pad)

    s_part = segsum(y, srcp, dstp, zrow)

    bz = bx_z + bh_z
    bh = bx_h + bh_h
    y2 = pl.pallas_call(
        _gates_body,
        grid=(grid,),
        in_specs=[
            pl.BlockSpec((BLK, d), lambda i: (i, 0)),
            pl.BlockSpec((NC, BLK, d), lambda i: (0, i, 0)),
            pl.BlockSpec((BLK, 1), lambda i: (i, 0)),
            pl.BlockSpec((BLK, 1), lambda i: (i, 0)),
            pl.BlockSpec(Wx_z.shape, lambda i: (0, 0, 0)),
            pl.BlockSpec(Wx_h.shape, lambda i: (0, 0, 0)),
            pl.BlockSpec(W_gcn.shape, lambda i: (0, 0)),
            pl.BlockSpec(bz.shape, lambda i: (0,)),
            pl.BlockSpec(bh.shape, lambda i: (0,)),
        ],
        out_specs=pl.BlockSpec((BLK, d), lambda i: (i, 0)),
        out_shape=jax.ShapeDtypeStruct((n_pad, d), jnp.float32),
    )(xp, s_part, dinv, dinv2, Wx_z, Wx_h, W_gcn, bz, bh)

    s2_part = segsum(y2, srcp, dstp, zrow)

    wl_row = W_lin.reshape(1, -1)
    outp = pl.pallas_call(
        _final_body,
        grid=(grid,),
        in_specs=[
            pl.BlockSpec((NC, BLK, d), lambda i: (0, i, 0)),
            pl.BlockSpec((BLK, d), lambda i: (i, 0)),
            pl.BlockSpec((BLK, 1), lambda i: (i, 0)),
            pl.BlockSpec(b_gcn.shape, lambda i: (0,)),
            pl.BlockSpec((1, d), lambda i: (0, 0)),
            pl.BlockSpec(b_lin.shape, lambda i: (0,)),
        ],
        out_specs=pl.BlockSpec((BLK, 1), lambda i: (i, 0)),
        out_shape=jax.ShapeDtypeStruct((n_pad, 1), jnp.float32),
    )(s2_part, y2, dinv2, b_gcn, wl_row, b_lin)

    return outp[:n, 0]


# speedup vs baseline: 47.0575x; 1.0387x over previous
import functools

import jax
import jax.numpy as jnp
from jax import lax
from jax.experimental import pallas as pl
from jax.experimental.pallas import tpu as pltpu
from jax.experimental.pallas import tpu_sc as plsc

NC = 2
NS = 16
NW = NC * NS
CHUNK = 128
BLK = 1280


def _mesh():
    return plsc.VectorSubcoreMesh(
        core_axis_name="c", subcore_axis_name="s", num_cores=NC, num_subcores=NS
    )


def _make_hist(n_pad, e_pad):
    epw = e_pad // NW
    nt = epw // CHUNK
    rpt = n_pad // NS

    @functools.partial(
        pl.kernel,
        out_type=(
            jax.ShapeDtypeStruct((NC, n_pad), jnp.float32),
            jax.ShapeDtypeStruct((NC, n_pad), jnp.float32),
        ),
        mesh=_mesh(),
        scratch_types=[
            pltpu.VMEM((nt, CHUNK), jnp.int32),
            pltpu.VMEM((nt, CHUNK), jnp.int32),
            pltpu.VMEM((CHUNK,), jnp.float32),
            pltpu.VMEM_SHARED((n_pad,), jnp.float32),
            pltpu.VMEM_SHARED((n_pad,), jnp.float32),
            pltpu.SemaphoreType.DMA((2,)),
        ],
    )
    def hist(src_hbm, dst_hbm, ones_hbm, z1_hbm, outs_hbm, outd_hbm,
             idx_s, idx_d, ones_v, acc_s, acc_d, sem):
        c = lax.axis_index("c")
        s = lax.axis_index("s")
        wid = s * NC + c
        r0 = s * rpt
        pltpu.sync_copy(ones_hbm, ones_v)
        pltpu.sync_copy(z1_hbm, acc_s.at[pl.ds(r0, rpt)])
        pltpu.sync_copy(z1_hbm, acc_d.at[pl.ds(r0, rpt)])
        pltpu.sync_copy(src_hbm.at[wid], idx_s)
        pltpu.sync_copy(dst_hbm.at[wid], idx_d)
        plsc.subcore_barrier()

        def body(i, carry):
            pltpu.async_copy(ones_v, acc_s.at[idx_s.at[i]], sem.at[0], add=True)
            pltpu.async_copy(ones_v, acc_d.at[idx_d.at[i]], sem.at[1], add=True)
            return carry

        lax.fori_loop(0, nt, body, 0)
        pltpu.make_async_copy(src_hbm.at[0], idx_s, sem.at[0]).wait()
        pltpu.make_async_copy(dst_hbm.at[0], idx_d, sem.at[1]).wait()
        plsc.subcore_barrier()
        pltpu.sync_copy(acc_s.at[pl.ds(r0, rpt)], outs_hbm.at[c, pl.ds(r0, rpt)])
        pltpu.sync_copy(acc_d.at[pl.ds(r0, rpt)], outd_hbm.at[c, pl.ds(r0, rpt)])

    return hist


def _make_segsum(n_pad, d, e_pad):
    epw = e_pad // NW
    nt = epw // CHUNK
    rpt = n_pad // NS
    NB = 2
    G = 16
    ngroups = nt // G

    @functools.partial(
        pl.kernel,
        out_type=jax.ShapeDtypeStruct((NC, n_pad, d), jnp.float32),
        mesh=_mesh(),
        scratch_types=[
            pltpu.VMEM((2, G, CHUNK), jnp.int32),
            pltpu.VMEM((2, G, CHUNK), jnp.int32),
            pltpu.VMEM((NB, CHUNK, d), jnp.float32),
            pltpu.VMEM_SHARED((n_pad, d), jnp.float32),
            pltpu.SemaphoreType.DMA((NB,)),
            pltpu.SemaphoreType.DMA((NB,)),
            pltpu.SemaphoreType.DMA((2,)),
        ],
    )
    def segsum(y_hbm, src_hbm, dst_hbm, zrow_hbm, out_hbm,
               idx_s, idx_d, rows, acc, gsem, ssem, isem):
        c = lax.axis_index("c")
        s = lax.axis_index("s")
        wid = s * NC + c
        r0 = s * rpt
        pltpu.sync_copy(zrow_hbm, acc.at[pl.ds(r0, rpt)])
        pltpu.sync_copy(src_hbm.at[wid, pl.ds(0, G)], idx_s.at[0])
        pltpu.sync_copy(dst_hbm.at[wid, pl.ds(0, G)], idx_d.at[0])
        plsc.subcore_barrier()

        pltpu.async_copy(y_hbm.at[idx_s.at[0, 0]], rows.at[0], gsem.at[0])

        def body(i, carry):
            g = lax.div(i, G)
            j = lax.rem(i, G)
            slot = lax.rem(g, 2)
            b = lax.rem(i, NB)
            pltpu.make_async_copy(
                y_hbm.at[pl.ds(0, CHUNK)], rows.at[b], gsem.at[b]).wait()
            pltpu.async_copy(rows.at[b], acc.at[idx_d.at[slot, j]],
                             ssem.at[b], add=True)

            @pl.when((j == 1) & (g + 1 < ngroups))
            def _():
                nslot = 1 - slot
                pltpu.async_copy(src_hbm.at[wid, pl.ds((g + 1) * G, G)],
                                 idx_s.at[nslot], isem.at[nslot])
                pltpu.async_copy(dst_hbm.at[wid, pl.ds((g + 1) * G, G)],
                                 idx_d.at[nslot], isem.at[nslot])

            @pl.when(i + 1 < nt)
            def _():
                b1 = lax.rem(i + 1, NB)

                @pl.when(i >= NB - 1)
                def _():
                    pltpu.make_async_copy(
                        y_hbm.at[pl.ds(0, CHUNK)], rows.at[b1],
                        ssem.at[b1]).wait()

                g1 = lax.div(i + 1, G)
                j1 = lax.rem(i + 1, G)
                slot1 = lax.rem(g1, 2)

                @pl.when(j1 == 0)
                def _():
                    pltpu.make_async_copy(
                        src_hbm.at[wid, pl.ds(0, G)], idx_s.at[slot1],
                        isem.at[slot1]).wait()
                    pltpu.make_async_copy(
                        dst_hbm.at[wid, pl.ds(0, G)], idx_d.at[slot1],
                        isem.at[slot1]).wait()

                pltpu.async_copy(y_hbm.at[idx_s.at[slot1, j1]], rows.at[b1],
                                 gsem.at[b1])

            return carry

        lax.fori_loop(0, nt, body, 0)
        for k in range(NB):
            bb = (nt - NB + k) % NB
            pltpu.make_async_copy(
                y_hbm.at[pl.ds(0, CHUNK)], rows.at[bb], ssem.at[bb]).wait()
        plsc.subcore_barrier()
        pltpu.sync_copy(acc.at[pl.ds(r0, rpt)], out_hbm.at[c, pl.ds(r0, rpt)])

    return segsum


def _scale_body(hs_ref, hd_ref, x_ref, y_ref, dinv_ref, dinv2_ref):
    deg = hs_ref[0, :] + hs_ref[1, :]
    deg2 = hd_ref[0, :] + hd_ref[1, :] + 1.0
    dinv = jnp.where(deg > 0.0, lax.rsqrt(deg), 0.0)
    dinv2 = lax.rsqrt(deg2)
    dinv_ref[...] = dinv[:, None]
    dinv2_ref[...] = dinv2[:, None]
    y_ref[...] = x_ref[...] * dinv[:, None]


def _gates_body(x_ref, s_ref, dinv_ref, dinv2_ref, wz_ref, wh_ref, wg_ref,
                bz_ref, bh_ref, y2_ref):
    tx1 = (s_ref[0] + s_ref[1]) * (-dinv_ref[...])
    xb = x_ref[...]
    az = (jnp.dot(xb, wz_ref[0], preferred_element_type=jnp.float32)
          + jnp.dot(tx1, wz_ref[1], preferred_element_type=jnp.float32)
          + bz_ref[...])
    ah = (jnp.dot(xb, wh_ref[0], preferred_element_type=jnp.float32)
          + jnp.dot(tx1, wh_ref[1], preferred_element_type=jnp.float32)
          + bh_ref[...])
    hn = (1.0 - jax.nn.sigmoid(az)) * jnp.tanh(ah)
    y2_ref[...] = jnp.dot(hn, wg_ref[...],
                          preferred_element_type=jnp.float32) * dinv2_ref[...]


def _final_body(s2_ref, y2_ref, dinv2_ref, bg_ref, wl_ref, bl_ref, out_ref):
    t = (s2_ref[0] + s2_ref[1] + y2_ref[...]) * dinv2_ref[...] + bg_ref[...]
    h1 = jnp.maximum(t, 0.0)
    out_ref[...] = (jnp.sum(h1 * wl_ref[...], axis=1) + bl_ref[0])[:, None]


def kernel(x, edge_index, Wx_z, bx_z, Wh_z, bh_z, Wx_r, bx_r, Wh_r, bh_r,
           Wx_h, bx_h, Wh_h, bh_h, W_gcn, b_gcn, W_lin, b_lin):
    n, d = x.shape
    e = edge_index.shape[1]
    n_pad = -(-n // BLK) * BLK
    e_pad = -(-e // (NW * CHUNK * 8)) * (NW * CHUNK * 8)
    grid = n_pad // BLK
    trash = n_pad - n

    pad_idx = n + (jnp.arange(e_pad - e, dtype=jnp.int32) % trash)
    srcp = jnp.concatenate([edge_index[0], pad_idx]).reshape(NW, -1, CHUNK)
    dstp = jnp.concatenate([edge_index[1], pad_idx]).reshape(NW, -1, CHUNK)
    xp = jnp.concatenate([x, jnp.zeros((trash, d), x.dtype)], axis=0)
    zrow = jnp.zeros((n_pad // NS, d), jnp.float32)
    z1 = jnp.zeros((n_pad // NS,), jnp.float32)
    ones_c = jnp.ones((CHUNK,), jnp.float32)

    hs, hd = _make_hist(n_pad, e_pad)(srcp, dstp, ones_c, z1)

    y, dinv, dinv2 = pl.pallas_call(
        _scale_body,
        grid=(grid,),
        in_specs=[
            pl.BlockSpec((NC, BLK), lambda i: (0, i)),
            pl.BlockSpec((NC, BLK), lambda i: (0, i)),
            pl.BlockSpec((BLK, d), lambda i: (i, 0)),
        ],
        out_specs=[
            pl.BlockSpec((BLK, d), lambda i: (i, 0)),
            pl.BlockSpec((BLK, 1), lambda i: (i, 0)),
            pl.BlockSpec((BLK, 1), lambda i: (i, 0)),
        ],
        out_shape=[
            jax.ShapeDtypeStruct((n_pad, d), jnp.float32),
            jax.ShapeDtypeStruct((n_pad, 1), jnp.float32),
            jax.ShapeDtypeStruct((n_pad, 1), jnp.float32),
        ],
    )(hs, hd, xp)

    segsum = _make_segsum(n_pad, d, e_pad)

    s_part = segsum(y, srcp, dstp, zrow)

    bz = bx_z + bh_z
    bh = bx_h + bh_h
    y2 = pl.pallas_call(
        _gates_body,
        grid=(grid,),
        in_specs=[
            pl.BlockSpec((BLK, d), lambda i: (i, 0)),
            pl.BlockSpec((NC, BLK, d), lambda i: (0, i, 0)),
            pl.BlockSpec((BLK, 1), lambda i: (i, 0)),
            pl.BlockSpec((BLK, 1), lambda i: (i, 0)),
            pl.BlockSpec(Wx_z.shape, lambda i: (0, 0, 0)),
            pl.BlockSpec(Wx_h.shape, lambda i: (0, 0, 0)),
            pl.BlockSpec(W_gcn.shape, lambda i: (0, 0)),
            pl.BlockSpec(bz.shape, lambda i: (0,)),
            pl.BlockSpec(bh.shape, lambda i: (0,)),
        ],
        out_specs=pl.BlockSpec((BLK, d), lambda i: (i, 0)),
        out_shape=jax.ShapeDtypeStruct((n_pad, d), jnp.float32),
    )(xp, s_part, dinv, dinv2, Wx_z, Wx_h, W_gcn, bz, bh)

    s2_part = segsum(y2, srcp, dstp, zrow)

    wl_row = W_lin.reshape(1, -1)
    outp = pl.pallas_call(
        _final_body,
        grid=(grid,),
        in_specs=[
            pl.BlockSpec((NC, BLK, d), lambda i: (0, i, 0)),
            pl.BlockSpec((BLK, d), lambda i: (i, 0)),
            pl.BlockSpec((BLK, 1), lambda i: (i, 0)),
            pl.BlockSpec(b_gcn.shape, lambda i: (0,)),
            pl.BlockSpec((1, d), lambda i: (0, 0)),
            pl.BlockSpec(b_lin.shape, lambda i: (0,)),
        ],
        out_specs=pl.BlockSpec((BLK, 1), lambda i: (i, 0)),
        out_shape=jax.ShapeDtypeStruct((n_pad, 1), jnp.float32),
    )(s2_part, y2, dinv2, b_gcn, wl_row, b_lin)

    return outp[:n, 0]


# speedup vs baseline: 48.8897x; 1.0389x over previous
import functools

import jax
import jax.numpy as jnp
from jax import lax
from jax.experimental import pallas as pl
from jax.experimental.pallas import tpu as pltpu
from jax.experimental.pallas import tpu_sc as plsc

NC = 2
NS = 16
NW = NC * NS
CHUNK = 128
BLK = 1280


def _mesh():
    return plsc.VectorSubcoreMesh(
        core_axis_name="c", subcore_axis_name="s", num_cores=NC, num_subcores=NS
    )


def _make_hist(n_pad, e_pad):
    epw = e_pad // NW
    nt = epw // CHUNK
    rpt = n_pad // NS

    @functools.partial(
        pl.kernel,
        out_type=(
            jax.ShapeDtypeStruct((NC, n_pad), jnp.float32),
            jax.ShapeDtypeStruct((NC, n_pad), jnp.float32),
        ),
        mesh=_mesh(),
        scratch_types=[
            pltpu.VMEM((nt, CHUNK), jnp.int32),
            pltpu.VMEM((nt, CHUNK), jnp.int32),
            pltpu.VMEM((CHUNK,), jnp.float32),
            pltpu.VMEM_SHARED((n_pad,), jnp.float32),
            pltpu.VMEM_SHARED((n_pad,), jnp.float32),
            pltpu.SemaphoreType.DMA((2,)),
        ],
    )
    def hist(src_hbm, dst_hbm, ones_hbm, z1_hbm, outs_hbm, outd_hbm,
             idx_s, idx_d, ones_v, acc_s, acc_d, sem):
        c = lax.axis_index("c")
        s = lax.axis_index("s")
        wid = s * NC + c
        r0 = s * rpt
        pltpu.sync_copy(ones_hbm, ones_v)
        pltpu.sync_copy(z1_hbm, acc_s.at[pl.ds(r0, rpt)])
        pltpu.sync_copy(z1_hbm, acc_d.at[pl.ds(r0, rpt)])
        pltpu.sync_copy(src_hbm.at[wid], idx_s)
        pltpu.sync_copy(dst_hbm.at[wid], idx_d)
        plsc.subcore_barrier()

        def body(i, carry):
            pltpu.async_copy(ones_v, acc_s.at[idx_s.at[i]], sem.at[0], add=True)
            pltpu.async_copy(ones_v, acc_d.at[idx_d.at[i]], sem.at[1], add=True)
            return carry

        lax.fori_loop(0, nt, body, 0)
        pltpu.make_async_copy(src_hbm.at[0], idx_s, sem.at[0]).wait()
        pltpu.make_async_copy(dst_hbm.at[0], idx_d, sem.at[1]).wait()
        plsc.subcore_barrier()
        pltpu.sync_copy(acc_s.at[pl.ds(r0, rpt)], outs_hbm.at[c, pl.ds(r0, rpt)])
        pltpu.sync_copy(acc_d.at[pl.ds(r0, rpt)], outd_hbm.at[c, pl.ds(r0, rpt)])

    return hist


def _make_segsum(n_pad, d, e_pad):
    SCH = 64
    NB = 4
    G = 16
    epw = e_pad // NW
    nt = epw // SCH
    ngroups = nt // G
    rpt = n_pad // NS

    @functools.partial(
        pl.kernel,
        out_type=jax.ShapeDtypeStruct((NC, n_pad, d), jnp.float32),
        mesh=_mesh(),
        scratch_types=[
            pltpu.VMEM((2, G, SCH), jnp.int32),
            pltpu.VMEM((2, G, SCH), jnp.int32),
            pltpu.VMEM((NB, SCH, d), jnp.float32),
            pltpu.VMEM_SHARED((n_pad, d), jnp.float32),
            pltpu.SemaphoreType.DMA((NB,)),
            pltpu.SemaphoreType.DMA((NB,)),
            pltpu.SemaphoreType.DMA((2,)),
        ],
    )
    def segsum(y_hbm, src_hbm, dst_hbm, zrow_hbm, out_hbm,
               idx_s, idx_d, rows, acc, gsem, ssem, isem):
        c = lax.axis_index("c")
        s = lax.axis_index("s")
        wid = s * NC + c
        r0 = s * rpt
        pltpu.sync_copy(zrow_hbm, acc.at[pl.ds(r0, rpt)])
        pltpu.sync_copy(src_hbm.at[wid, pl.ds(0, G)], idx_s.at[0])
        pltpu.sync_copy(dst_hbm.at[wid, pl.ds(0, G)], idx_d.at[0])
        plsc.subcore_barrier()

        pltpu.async_copy(y_hbm.at[idx_s.at[0, 0]], rows.at[0], gsem.at[0])
        pltpu.async_copy(y_hbm.at[idx_s.at[0, 1]], rows.at[1], gsem.at[1])

        def body(i, carry):
            g = lax.div(i, G)
            j = lax.rem(i, G)
            slot = lax.rem(g, 2)
            b = lax.rem(i, NB)
            pltpu.make_async_copy(
                y_hbm.at[pl.ds(0, SCH)], rows.at[b], gsem.at[b]).wait()
            pltpu.async_copy(rows.at[b], acc.at[idx_d.at[slot, j]],
                             ssem.at[b], add=True)

            @pl.when((j == 2) & (g + 1 < ngroups))
            def _():
                nslot = 1 - slot
                pltpu.async_copy(src_hbm.at[wid, pl.ds((g + 1) * G, G)],
                                 idx_s.at[nslot], isem.at[nslot])
                pltpu.async_copy(dst_hbm.at[wid, pl.ds((g + 1) * G, G)],
                                 idx_d.at[nslot], isem.at[nslot])

            @pl.when(i + 2 < nt)
            def _():
                b2 = lax.rem(i + 2, NB)

                @pl.when(i >= NB - 2)
                def _():
                    pltpu.make_async_copy(
                        y_hbm.at[pl.ds(0, SCH)], rows.at[b2],
                        ssem.at[b2]).wait()

                g2 = lax.div(i + 2, G)
                j2 = lax.rem(i + 2, G)
                slot2 = lax.rem(g2, 2)

                @pl.when(j2 == 0)
                def _():
                    pltpu.make_async_copy(
                        src_hbm.at[wid, pl.ds(0, G)], idx_s.at[slot2],
                        isem.at[slot2]).wait()
                    pltpu.make_async_copy(
                        dst_hbm.at[wid, pl.ds(0, G)], idx_d.at[slot2],
                        isem.at[slot2]).wait()

                pltpu.async_copy(y_hbm.at[idx_s.at[slot2, j2]], rows.at[b2],
                                 gsem.at[b2])

            return carry

        lax.fori_loop(0, nt, body, 0)
        for k in range(NB):
            bb = (nt - NB + k) % NB
            pltpu.make_async_copy(
                y_hbm.at[pl.ds(0, SCH)], rows.at[bb], ssem.at[bb]).wait()
        plsc.subcore_barrier()
        pltpu.sync_copy(acc.at[pl.ds(r0, rpt)], out_hbm.at[c, pl.ds(r0, rpt)])

    return segsum


def _scale_body(hs_ref, hd_ref, x_ref, y_ref, dinv_ref, dinv2_ref):
    deg = hs_ref[0, :] + hs_ref[1, :]
    deg2 = hd_ref[0, :] + hd_ref[1, :] + 1.0
    dinv = jnp.where(deg > 0.0, lax.rsqrt(deg), 0.0)
    dinv2 = lax.rsqrt(deg2)
    dinv_ref[...] = dinv[:, None]
    dinv2_ref[...] = dinv2[:, None]
    y_ref[...] = x_ref[...] * dinv[:, None]


def _gates_body(x_ref, s_ref, dinv_ref, dinv2_ref, wz_ref, wh_ref, wg_ref,
                bz_ref, bh_ref, y2_ref):
    tx1 = (s_ref[0] + s_ref[1]) * (-dinv_ref[...])
    xb = x_ref[...]
    az = (jnp.dot(xb, wz_ref[0], preferred_element_type=jnp.float32)
          + jnp.dot(tx1, wz_ref[1], preferred_element_type=jnp.float32)
          + bz_ref[...])
    ah = (jnp.dot(xb, wh_ref[0], preferred_element_type=jnp.float32)
          + jnp.dot(tx1, wh_ref[1], preferred_element_type=jnp.float32)
          + bh_ref[...])
    hn = (1.0 - jax.nn.sigmoid(az)) * jnp.tanh(ah)
    y2_ref[...] = jnp.dot(hn, wg_ref[...],
                          preferred_element_type=jnp.float32) * dinv2_ref[...]


def _final_body(s2_ref, y2_ref, dinv2_ref, bg_ref, wl_ref, bl_ref, out_ref):
    t = (s2_ref[0] + s2_ref[1] + y2_ref[...]) * dinv2_ref[...] + bg_ref[...]
    h1 = jnp.maximum(t, 0.0)
    out_ref[...] = (jnp.sum(h1 * wl_ref[...], axis=1) + bl_ref[0])[:, None]


def kernel(x, edge_index, Wx_z, bx_z, Wh_z, bh_z, Wx_r, bx_r, Wh_r, bh_r,
           Wx_h, bx_h, Wh_h, bh_h, W_gcn, b_gcn, W_lin, b_lin):
    n, d = x.shape
    e = edge_index.shape[1]
    n_pad = -(-n // BLK) * BLK
    e_pad = -(-e // (NW * CHUNK * 8)) * (NW * CHUNK * 8)
    grid = n_pad // BLK
    trash = n_pad - n

    pad_idx = n + (jnp.arange(e_pad - e, dtype=jnp.int32) % trash)
    src_flat = jnp.concatenate([edge_index[0], pad_idx])
    dst_flat = jnp.concatenate([edge_index[1], pad_idx])
    srcp = src_flat.reshape(NW, -1, CHUNK)
    dstp = dst_flat.reshape(NW, -1, CHUNK)
    src64 = src_flat.reshape(NW, -1, 64)
    dst64 = dst_flat.reshape(NW, -1, 64)
    xp = jnp.concatenate([x, jnp.zeros((trash, d), x.dtype)], axis=0)
    zrow = jnp.zeros((n_pad // NS, d), jnp.float32)
    z1 = jnp.zeros((n_pad // NS,), jnp.float32)
    ones_c = jnp.ones((CHUNK,), jnp.float32)

    hs, hd = _make_hist(n_pad, e_pad)(srcp, dstp, ones_c, z1)

    y, dinv, dinv2 = pl.pallas_call(
        _scale_body,
        grid=(grid,),
        in_specs=[
            pl.BlockSpec((NC, BLK), lambda i: (0, i)),
            pl.BlockSpec((NC, BLK), lambda i: (0, i)),
            pl.BlockSpec((BLK, d), lambda i: (i, 0)),
        ],
        out_specs=[
            pl.BlockSpec((BLK, d), lambda i: (i, 0)),
            pl.BlockSpec((BLK, 1), lambda i: (i, 0)),
            pl.BlockSpec((BLK, 1), lambda i: (i, 0)),
        ],
        out_shape=[
            jax.ShapeDtypeStruct((n_pad, d), jnp.float32),
            jax.ShapeDtypeStruct((n_pad, 1), jnp.float32),
            jax.ShapeDtypeStruct((n_pad, 1), jnp.float32),
        ],
    )(hs, hd, xp)

    segsum = _make_segsum(n_pad, d, e_pad)

    s_part = segsum(y, src64, dst64, zrow)

    bz = bx_z + bh_z
    bh = bx_h + bh_h
    y2 = pl.pallas_call(
        _gates_body,
        grid=(grid,),
        in_specs=[
            pl.BlockSpec((BLK, d), lambda i: (i, 0)),
            pl.BlockSpec((NC, BLK, d), lambda i: (0, i, 0)),
            pl.BlockSpec((BLK, 1), lambda i: (i, 0)),
            pl.BlockSpec((BLK, 1), lambda i: (i, 0)),
            pl.BlockSpec(Wx_z.shape, lambda i: (0, 0, 0)),
            pl.BlockSpec(Wx_h.shape, lambda i: (0, 0, 0)),
            pl.BlockSpec(W_gcn.shape, lambda i: (0, 0)),
            pl.BlockSpec(bz.shape, lambda i: (0,)),
            pl.BlockSpec(bh.shape, lambda i: (0,)),
        ],
        out_specs=pl.BlockSpec((BLK, d), lambda i: (i, 0)),
        out_shape=jax.ShapeDtypeStruct((n_pad, d), jnp.float32),
    )(xp, s_part, dinv, dinv2, Wx_z, Wx_h, W_gcn, bz, bh)

    s2_part = segsum(y2, src64, dst64, zrow)

    wl_row = W_lin.reshape(1, -1)
    outp = pl.pallas_call(
        _final_body,
        grid=(grid,),
        in_specs=[
            pl.BlockSpec((NC, BLK, d), lambda i: (0, i, 0)),
            pl.BlockSpec((BLK, d), lambda i: (i, 0)),
            pl.BlockSpec((BLK, 1), lambda i: (i, 0)),
            pl.BlockSpec(b_gcn.shape, lambda i: (0,)),
            pl.BlockSpec((1, d), lambda i: (0, 0)),
            pl.BlockSpec(b_lin.shape, lambda i: (0,)),
        ],
        out_specs=pl.BlockSpec((BLK, 1), lambda i: (i, 0)),
        out_shape=jax.ShapeDtypeStruct((n_pad, 1), jnp.float32),
    )(s2_part, y2, dinv2, b_gcn, wl_row, b_lin)

    return outp[:n, 0]


# speedup vs baseline: 54.5524x; 1.1158x over previous
import functools

import jax
import jax.numpy as jnp
from jax import lax
from jax.experimental import pallas as pl
from jax.experimental.pallas import tpu as pltpu
from jax.experimental.pallas import tpu_sc as plsc

NC = 2
NS = 16
NW = NC * NS
CHUNK = 128
BLK = 1280


def _mesh():
    return plsc.VectorSubcoreMesh(
        core_axis_name="c", subcore_axis_name="s", num_cores=NC, num_subcores=NS
    )


def _make_hist(n_pad, e_pad):
    epw = e_pad // NW
    nt = epw // CHUNK
    rpt = n_pad // NS

    @functools.partial(
        pl.kernel,
        out_type=(
            jax.ShapeDtypeStruct((NC, n_pad), jnp.float32),
            jax.ShapeDtypeStruct((NC, n_pad), jnp.float32),
        ),
        mesh=_mesh(),
        scratch_types=[
            pltpu.VMEM((nt, CHUNK), jnp.int32),
            pltpu.VMEM((nt, CHUNK), jnp.int32),
            pltpu.VMEM((CHUNK,), jnp.float32),
            pltpu.VMEM_SHARED((n_pad,), jnp.float32),
            pltpu.VMEM_SHARED((n_pad,), jnp.float32),
            pltpu.SemaphoreType.DMA((2,)),
        ],
    )
    def hist(src_hbm, dst_hbm, ones_hbm, z1_hbm, outs_hbm, outd_hbm,
             idx_s, idx_d, ones_v, acc_s, acc_d, sem):
        c = lax.axis_index("c")
        s = lax.axis_index("s")
        wid = s * NC + c
        r0 = s * rpt
        pltpu.sync_copy(ones_hbm, ones_v)
        pltpu.sync_copy(z1_hbm, acc_s.at[pl.ds(r0, rpt)])
        pltpu.sync_copy(z1_hbm, acc_d.at[pl.ds(r0, rpt)])
        pltpu.sync_copy(src_hbm.at[wid], idx_s)
        pltpu.sync_copy(dst_hbm.at[wid], idx_d)
        plsc.subcore_barrier()

        def body(i, carry):
            pltpu.async_copy(ones_v, acc_s.at[idx_s.at[i]], sem.at[0], add=True)
            pltpu.async_copy(ones_v, acc_d.at[idx_d.at[i]], sem.at[1], add=True)
            return carry

        lax.fori_loop(0, nt, body, 0)
        pltpu.make_async_copy(src_hbm.at[0], idx_s, sem.at[0]).wait()
        pltpu.make_async_copy(dst_hbm.at[0], idx_d, sem.at[1]).wait()
        plsc.subcore_barrier()
        pltpu.sync_copy(acc_s.at[pl.ds(r0, rpt)], outs_hbm.at[c, pl.ds(r0, rpt)])
        pltpu.sync_copy(acc_d.at[pl.ds(r0, rpt)], outd_hbm.at[c, pl.ds(r0, rpt)])

    return hist


def _make_segsum(n_pad, d, e_pad):
    SCH = 64
    NB = 4
    G = 16
    epw = e_pad // NW
    nt = epw // SCH
    ngroups = nt // G
    rpt = n_pad // NS

    @functools.partial(
        pl.kernel,
        out_type=jax.ShapeDtypeStruct((NC, n_pad, d), jnp.float32),
        mesh=_mesh(),
        scratch_types=[
            pltpu.VMEM((2, G, SCH), jnp.int32),
            pltpu.VMEM((2, G, SCH), jnp.int32),
            pltpu.VMEM((NB, SCH, d), jnp.float32),
            pltpu.VMEM_SHARED((n_pad, d), jnp.float32),
            pltpu.SemaphoreType.DMA((NB,)),
            pltpu.SemaphoreType.DMA((NB,)),
            pltpu.SemaphoreType.DMA((2,)),
        ],
    )
    def segsum(y_hbm, src_hbm, dst_hbm, zrow_hbm, out_hbm,
               idx_s, idx_d, rows, acc, gsem, ssem, isem):
        c = lax.axis_index("c")
        s = lax.axis_index("s")
        wid = s * NC + c
        r0 = s * rpt
        pltpu.sync_copy(zrow_hbm, acc.at[pl.ds(r0, rpt)])
        pltpu.sync_copy(src_hbm.at[wid, pl.ds(0, G)], idx_s.at[0])
        pltpu.sync_copy(dst_hbm.at[wid, pl.ds(0, G)], idx_d.at[0])
        plsc.subcore_barrier()

        pltpu.async_copy(y_hbm.at[idx_s.at[0, 0]], rows.at[0], gsem.at[0])
        pltpu.async_copy(y_hbm.at[idx_s.at[0, 1]], rows.at[1], gsem.at[1])
        pltpu.async_copy(y_hbm.at[idx_s.at[0, 2]], rows.at[2], gsem.at[2])

        def body(i, carry):
            g = lax.div(i, G)
            j = lax.rem(i, G)
            slot = lax.rem(g, 2)
            b = lax.rem(i, NB)
            pltpu.make_async_copy(
                y_hbm.at[pl.ds(0, SCH)], rows.at[b], gsem.at[b]).wait()
            pltpu.async_copy(rows.at[b], acc.at[idx_d.at[slot, j]],
                             ssem.at[b], add=True)

            @pl.when((j == 2) & (g + 1 < ngroups))
            def _():
                nslot = 1 - slot
                pltpu.async_copy(src_hbm.at[wid, pl.ds((g + 1) * G, G)],
                                 idx_s.at[nslot], isem.at[nslot])
                pltpu.async_copy(dst_hbm.at[wid, pl.ds((g + 1) * G, G)],
                                 idx_d.at[nslot], isem.at[nslot])

            @pl.when(i + 3 < nt)
            def _():
                b2 = lax.rem(i + 3, NB)

                @pl.when(i >= 1)
                def _():
                    pltpu.make_async_copy(
                        y_hbm.at[pl.ds(0, SCH)], rows.at[b2],
                        ssem.at[b2]).wait()

                g2 = lax.div(i + 3, G)
                j2 = lax.rem(i + 3, G)
                slot2 = lax.rem(g2, 2)

                @pl.when(j2 == 0)
                def _():
                    pltpu.make_async_copy(
                        src_hbm.at[wid, pl.ds(0, G)], idx_s.at[slot2],
                        isem.at[slot2]).wait()
                    pltpu.make_async_copy(
                        dst_hbm.at[wid, pl.ds(0, G)], idx_d.at[slot2],
                        isem.at[slot2]).wait()

                pltpu.async_copy(y_hbm.at[idx_s.at[slot2, j2]], rows.at[b2],
                                 gsem.at[b2])

            return carry

        lax.fori_loop(0, nt, body, 0)
        for k in range(NB):
            bb = (nt - NB + k) % NB
            pltpu.make_async_copy(
                y_hbm.at[pl.ds(0, SCH)], rows.at[bb], ssem.at[bb]).wait()
        plsc.subcore_barrier()
        pltpu.sync_copy(acc.at[pl.ds(r0, rpt)], out_hbm.at[c, pl.ds(r0, rpt)])

    return segsum


def _scale_body(hs_ref, hd_ref, x_ref, y_ref, dinv_ref, dinv2_ref):
    deg = hs_ref[0, :] + hs_ref[1, :]
    deg2 = hd_ref[0, :] + hd_ref[1, :] + 1.0
    dinv = jnp.where(deg > 0.0, lax.rsqrt(deg), 0.0)
    dinv2 = lax.rsqrt(deg2)
    dinv_ref[...] = dinv[:, None]
    dinv2_ref[...] = dinv2[:, None]
    y_ref[...] = x_ref[...] * dinv[:, None]


def _gates_body(x_ref, s_ref, dinv_ref, dinv2_ref, wz_ref, wh_ref, wg_ref,
                bz_ref, bh_ref, y2_ref):
    tx1 = (s_ref[0] + s_ref[1]) * (-dinv_ref[...])
    xb = x_ref[...]
    az = (jnp.dot(xb, wz_ref[0], preferred_element_type=jnp.float32)
          + jnp.dot(tx1, wz_ref[1], preferred_element_type=jnp.float32)
          + bz_ref[...])
    ah = (jnp.dot(xb, wh_ref[0], preferred_element_type=jnp.float32)
          + jnp.dot(tx1, wh_ref[1], preferred_element_type=jnp.float32)
          + bh_ref[...])
    hn = (1.0 - jax.nn.sigmoid(az)) * jnp.tanh(ah)
    y2_ref[...] = jnp.dot(hn, wg_ref[...],
                          preferred_element_type=jnp.float32) * dinv2_ref[...]


def _final_body(s2_ref, y2_ref, dinv2_ref, bg_ref, wl_ref, bl_ref, out_ref):
    t = (s2_ref[0] + s2_ref[1] + y2_ref[...]) * dinv2_ref[...] + bg_ref[...]
    h1 = jnp.maximum(t, 0.0)
    out_ref[...] = (jnp.sum(h1 * wl_ref[...], axis=1) + bl_ref[0])[:, None]


def kernel(x, edge_index, Wx_z, bx_z, Wh_z, bh_z, Wx_r, bx_r, Wh_r, bh_r,
           Wx_h, bx_h, Wh_h, bh_h, W_gcn, b_gcn, W_lin, b_lin):
    n, d = x.shape
    e = edge_index.shape[1]
    n_pad = -(-n // BLK) * BLK
    e_pad = -(-e // (NW * CHUNK * 8)) * (NW * CHUNK * 8)
    grid = n_pad // BLK
    trash = n_pad - n

    pad_idx = n + (jnp.arange(e_pad - e, dtype=jnp.int32) % trash)
    src_flat = jnp.concatenate([edge_index[0], pad_idx])
    dst_flat = jnp.concatenate([edge_index[1], pad_idx])
    srcp = src_flat.reshape(NW, -1, CHUNK)
    dstp = dst_flat.reshape(NW, -1, CHUNK)
    src64 = src_flat.reshape(NW, -1, 64)
    dst64 = dst_flat.reshape(NW, -1, 64)
    xp = jnp.concatenate([x, jnp.zeros((trash, d), x.dtype)], axis=0)
    zrow = jnp.zeros((n_pad // NS, d), jnp.float32)
    z1 = jnp.zeros((n_pad // NS,), jnp.float32)
    ones_c = jnp.ones((CHUNK,), jnp.float32)

    hs, hd = _make_hist(n_pad, e_pad)(srcp, dstp, ones_c, z1)

    y, dinv, dinv2 = pl.pallas_call(
        _scale_body,
        grid=(grid,),
        in_specs=[
            pl.BlockSpec((NC, BLK), lambda i: (0, i)),
            pl.BlockSpec((NC, BLK), lambda i: (0, i)),
            pl.BlockSpec((BLK, d), lambda i: (i, 0)),
        ],
        out_specs=[
            pl.BlockSpec((BLK, d), lambda i: (i, 0)),
            pl.BlockSpec((BLK, 1), lambda i: (i, 0)),
            pl.BlockSpec((BLK, 1), lambda i: (i, 0)),
        ],
        out_shape=[
            jax.ShapeDtypeStruct((n_pad, d), jnp.float32),
            jax.ShapeDtypeStruct((n_pad, 1), jnp.float32),
            jax.ShapeDtypeStruct((n_pad, 1), jnp.float32),
        ],
    )(hs, hd, xp)

    segsum = _make_segsum(n_pad, d, e_pad)

    s_part = segsum(y, src64, dst64, zrow)

    bz = bx_z + bh_z
    bh = bx_h + bh_h
    y2 = pl.pallas_call(
        _gates_body,
        grid=(grid,),
        in_specs=[
            pl.BlockSpec((BLK, d), lambda i: (i, 0)),
            pl.BlockSpec((NC, BLK, d), lambda i: (0, i, 0)),
            pl.BlockSpec((BLK, 1), lambda i: (i, 0)),
            pl.BlockSpec((BLK, 1), lambda i: (i, 0)),
            pl.BlockSpec(Wx_z.shape, lambda i: (0, 0, 0)),
            pl.BlockSpec(Wx_h.shape, lambda i: (0, 0, 0)),
            pl.BlockSpec(W_gcn.shape, lambda i: (0, 0)),
            pl.BlockSpec(bz.shape, lambda i: (0,)),
            pl.BlockSpec(bh.shape, lambda i: (0,)),
        ],
        out_specs=pl.BlockSpec((BLK, d), lambda i: (i, 0)),
        out_shape=jax.ShapeDtypeStruct((n_pad, d), jnp.float32),
    )(xp, s_part, dinv, dinv2, Wx_z, Wx_h, W_gcn, bz, bh)

    s2_part = segsum(y2, src64, dst64, zrow)

    wl_row = W_lin.reshape(1, -1)
    outp = pl.pallas_call(
        _final_body,
        grid=(grid,),
        in_specs=[
            pl.BlockSpec((NC, BLK, d), lambda i: (0, i, 0)),
            pl.BlockSpec((BLK, d), lambda i: (i, 0)),
            pl.BlockSpec((BLK, 1), lambda i: (i, 0)),
            pl.BlockSpec(b_gcn.shape, lambda i: (0,)),
            pl.BlockSpec((1, d), lambda i: (0, 0)),
            pl.BlockSpec(b_lin.shape, lambda i: (0,)),
        ],
        out_specs=pl.BlockSpec((BLK, 1), lambda i: (i, 0)),
        out_shape=jax.ShapeDtypeStruct((n_pad, 1), jnp.float32),
    )(s2_part, y2, dinv2, b_gcn, wl_row, b_lin)

    return outp[:n, 0]


# speedup vs baseline: 57.1547x; 1.0477x over previous
import functools

import jax
import jax.numpy as jnp
from jax import lax
from jax.experimental import pallas as pl
from jax.experimental.pallas import tpu as pltpu
from jax.experimental.pallas import tpu_sc as plsc

NC = 2
NS = 16
NW = NC * NS
CHUNK = 128
BLK = 1280


def _mesh():
    return plsc.VectorSubcoreMesh(
        core_axis_name="c", subcore_axis_name="s", num_cores=NC, num_subcores=NS
    )


def _make_hist(n_pad, e_pad):
    epw = e_pad // NW
    nt = epw // CHUNK
    rpt = n_pad // NS

    @functools.partial(
        pl.kernel,
        out_type=(
            jax.ShapeDtypeStruct((NC, n_pad), jnp.float32),
            jax.ShapeDtypeStruct((NC, n_pad), jnp.float32),
        ),
        mesh=_mesh(),
        scratch_types=[
            pltpu.VMEM((nt, CHUNK), jnp.int32),
            pltpu.VMEM((nt, CHUNK), jnp.int32),
            pltpu.VMEM((CHUNK,), jnp.float32),
            pltpu.VMEM_SHARED((n_pad,), jnp.float32),
            pltpu.VMEM_SHARED((n_pad,), jnp.float32),
            pltpu.SemaphoreType.DMA((2,)),
        ],
    )
    def hist(src_hbm, dst_hbm, ones_hbm, z1_hbm, outs_hbm, outd_hbm,
             idx_s, idx_d, ones_v, acc_s, acc_d, sem):
        c = lax.axis_index("c")
        s = lax.axis_index("s")
        wid = s * NC + c
        r0 = s * rpt
        pltpu.sync_copy(ones_hbm, ones_v)
        pltpu.sync_copy(z1_hbm, acc_s.at[pl.ds(r0, rpt)])
        pltpu.sync_copy(z1_hbm, acc_d.at[pl.ds(r0, rpt)])
        pltpu.sync_copy(src_hbm.at[wid], idx_s)
        pltpu.sync_copy(dst_hbm.at[wid], idx_d)
        plsc.subcore_barrier()

        def body(i, carry):
            pltpu.async_copy(ones_v, acc_s.at[idx_s.at[i]], sem.at[0], add=True)
            pltpu.async_copy(ones_v, acc_d.at[idx_d.at[i]], sem.at[1], add=True)
            return carry

        lax.fori_loop(0, nt, body, 0)
        pltpu.make_async_copy(src_hbm.at[0], idx_s, sem.at[0]).wait()
        pltpu.make_async_copy(dst_hbm.at[0], idx_d, sem.at[1]).wait()
        plsc.subcore_barrier()
        pltpu.sync_copy(acc_s.at[pl.ds(r0, rpt)], outs_hbm.at[c, pl.ds(r0, rpt)])
        pltpu.sync_copy(acc_d.at[pl.ds(r0, rpt)], outd_hbm.at[c, pl.ds(r0, rpt)])

    return hist


def _make_segsum(n_pad, d, e_pad):
    SCH = 64
    NB = 5
    G = 16
    epw = e_pad // NW
    nt = epw // SCH
    ngroups = nt // G
    rpt = n_pad // NS

    @functools.partial(
        pl.kernel,
        out_type=jax.ShapeDtypeStruct((NC, n_pad, d), jnp.float32),
        mesh=_mesh(),
        scratch_types=[
            pltpu.VMEM((2, G, SCH), jnp.int32),
            pltpu.VMEM((2, G, SCH), jnp.int32),
            pltpu.VMEM((NB, SCH, d), jnp.float32),
            pltpu.VMEM_SHARED((n_pad, d), jnp.float32),
            pltpu.SemaphoreType.DMA((NB,)),
            pltpu.SemaphoreType.DMA((NB,)),
            pltpu.SemaphoreType.DMA((2,)),
        ],
    )
    def segsum(y_hbm, src_hbm, dst_hbm, zrow_hbm, out_hbm,
               idx_s, idx_d, rows, acc, gsem, ssem, isem):
        c = lax.axis_index("c")
        s = lax.axis_index("s")
        wid = s * NC + c
        r0 = s * rpt
        pltpu.sync_copy(zrow_hbm, acc.at[pl.ds(r0, rpt)])
        pltpu.sync_copy(src_hbm.at[wid, pl.ds(0, G)], idx_s.at[0])
        pltpu.sync_copy(dst_hbm.at[wid, pl.ds(0, G)], idx_d.at[0])
        plsc.subcore_barrier()

        pltpu.async_copy(y_hbm.at[idx_s.at[0, 0]], rows.at[0], gsem.at[0])
        pltpu.async_copy(y_hbm.at[idx_s.at[0, 1]], rows.at[1], gsem.at[1])
        pltpu.async_copy(y_hbm.at[idx_s.at[0, 2]], rows.at[2], gsem.at[2])
        pltpu.async_copy(y_hbm.at[idx_s.at[0, 3]], rows.at[3], gsem.at[3])

        def body(i, carry):
            g = lax.div(i, G)
            j = lax.rem(i, G)
            slot = lax.rem(g, 2)
            b = lax.rem(i, NB)
            pltpu.make_async_copy(
                y_hbm.at[pl.ds(0, SCH)], rows.at[b], gsem.at[b]).wait()
            pltpu.async_copy(rows.at[b], acc.at[idx_d.at[slot, j]],
                             ssem.at[b], add=True)

            @pl.when((j == 2) & (g + 1 < ngroups))
            def _():
                nslot = 1 - slot
                pltpu.async_copy(src_hbm.at[wid, pl.ds((g + 1) * G, G)],
                                 idx_s.at[nslot], isem.at[nslot])
                pltpu.async_copy(dst_hbm.at[wid, pl.ds((g + 1) * G, G)],
                                 idx_d.at[nslot], isem.at[nslot])

            @pl.when(i + 4 < nt)
            def _():
                b2 = lax.rem(i + 4, NB)

                @pl.when(i >= 1)
                def _():
                    pltpu.make_async_copy(
                        y_hbm.at[pl.ds(0, SCH)], rows.at[b2],
                        ssem.at[b2]).wait()

                g2 = lax.div(i + 4, G)
                j2 = lax.rem(i + 4, G)
                slot2 = lax.rem(g2, 2)

                @pl.when(j2 == 0)
                def _():
                    pltpu.make_async_copy(
                        src_hbm.at[wid, pl.ds(0, G)], idx_s.at[slot2],
                        isem.at[slot2]).wait()
                    pltpu.make_async_copy(
                        dst_hbm.at[wid, pl.ds(0, G)], idx_d.at[slot2],
                        isem.at[slot2]).wait()

                pltpu.async_copy(y_hbm.at[idx_s.at[slot2, j2]], rows.at[b2],
                                 gsem.at[b2])

            return carry

        lax.fori_loop(0, nt, body, 0)
        for k in range(NB):
            bb = (nt - NB + k) % NB
            pltpu.make_async_copy(
                y_hbm.at[pl.ds(0, SCH)], rows.at[bb], ssem.at[bb]).wait()
        plsc.subcore_barrier()
        pltpu.sync_copy(acc.at[pl.ds(r0, rpt)], out_hbm.at[c, pl.ds(r0, rpt)])

    return segsum


def _scale_body(hs_ref, hd_ref, x_ref, y_ref, dinv_ref, dinv2_ref):
    deg = hs_ref[0, :] + hs_ref[1, :]
    deg2 = hd_ref[0, :] + hd_ref[1, :] + 1.0
    dinv = jnp.where(deg > 0.0, lax.rsqrt(deg), 0.0)
    dinv2 = lax.rsqrt(deg2)
    dinv_ref[...] = dinv[:, None]
    dinv2_ref[...] = dinv2[:, None]
    y_ref[...] = x_ref[...] * dinv[:, None]


def _gates_body(x_ref, s_ref, dinv_ref, dinv2_ref, wz_ref, wh_ref, wg_ref,
                bz_ref, bh_ref, y2_ref):
    tx1 = (s_ref[0] + s_ref[1]) * (-dinv_ref[...])
    xb = x_ref[...]
    az = (jnp.dot(xb, wz_ref[0], preferred_element_type=jnp.float32)
          + jnp.dot(tx1, wz_ref[1], preferred_element_type=jnp.float32)
          + bz_ref[...])
    ah = (jnp.dot(xb, wh_ref[0], preferred_element_type=jnp.float32)
          + jnp.dot(tx1, wh_ref[1], preferred_element_type=jnp.float32)
          + bh_ref[...])
    hn = (1.0 - jax.nn.sigmoid(az)) * jnp.tanh(ah)
    y2_ref[...] = jnp.dot(hn, wg_ref[...],
                          preferred_element_type=jnp.float32) * dinv2_ref[...]


def _final_body(s2_ref, y2_ref, dinv2_ref, bg_ref, wl_ref, bl_ref, out_ref):
    t = (s2_ref[0] + s2_ref[1] + y2_ref[...]) * dinv2_ref[...] + bg_ref[...]
    h1 = jnp.maximum(t, 0.0)
    out_ref[...] = (jnp.sum(h1 * wl_ref[...], axis=1) + bl_ref[0])[:, None]


def kernel(x, edge_index, Wx_z, bx_z, Wh_z, bh_z, Wx_r, bx_r, Wh_r, bh_r,
           Wx_h, bx_h, Wh_h, bh_h, W_gcn, b_gcn, W_lin, b_lin):
    n, d = x.shape
    e = edge_index.shape[1]
    n_pad = -(-n // BLK) * BLK
    e_pad = -(-e // (NW * CHUNK * 8)) * (NW * CHUNK * 8)
    grid = n_pad // BLK
    trash = n_pad - n

    pad_idx = n + (jnp.arange(e_pad - e, dtype=jnp.int32) % trash)
    src_flat = jnp.concatenate([edge_index[0], pad_idx])
    dst_flat = jnp.concatenate([edge_index[1], pad_idx])
    srcp = src_flat.reshape(NW, -1, CHUNK)
    dstp = dst_flat.reshape(NW, -1, CHUNK)
    src64 = src_flat.reshape(NW, -1, 64)
    dst64 = dst_flat.reshape(NW, -1, 64)
    xp = jnp.concatenate([x, jnp.zeros((trash, d), x.dtype)], axis=0)
    zrow = jnp.zeros((n_pad // NS, d), jnp.float32)
    z1 = jnp.zeros((n_pad // NS,), jnp.float32)
    ones_c = jnp.ones((CHUNK,), jnp.float32)

    hs, hd = _make_hist(n_pad, e_pad)(srcp, dstp, ones_c, z1)

    y, dinv, dinv2 = pl.pallas_call(
        _scale_body,
        grid=(grid,),
        in_specs=[
            pl.BlockSpec((NC, BLK), lambda i: (0, i)),
            pl.BlockSpec((NC, BLK), lambda i: (0, i)),
            pl.BlockSpec((BLK, d), lambda i: (i, 0)),
        ],
        out_specs=[
            pl.BlockSpec((BLK, d), lambda i: (i, 0)),
            pl.BlockSpec((BLK, 1), lambda i: (i, 0)),
            pl.BlockSpec((BLK, 1), lambda i: (i, 0)),
        ],
        out_shape=[
            jax.ShapeDtypeStruct((n_pad, d), jnp.float32),
            jax.ShapeDtypeStruct((n_pad, 1), jnp.float32),
            jax.ShapeDtypeStruct((n_pad, 1), jnp.float32),
        ],
    )(hs, hd, xp)

    segsum = _make_segsum(n_pad, d, e_pad)

    s_part = segsum(y, src64, dst64, zrow)

    bz = bx_z + bh_z
    bh = bx_h + bh_h
    y2 = pl.pallas_call(
        _gates_body,
        grid=(grid,),
        in_specs=[
            pl.BlockSpec((BLK, d), lambda i: (i, 0)),
            pl.BlockSpec((NC, BLK, d), lambda i: (0, i, 0)),
            pl.BlockSpec((BLK, 1), lambda i: (i, 0)),
            pl.BlockSpec((BLK, 1), lambda i: (i, 0)),
            pl.BlockSpec(Wx_z.shape, lambda i: (0, 0, 0)),
            pl.BlockSpec(Wx_h.shape, lambda i: (0, 0, 0)),
            pl.BlockSpec(W_gcn.shape, lambda i: (0, 0)),
            pl.BlockSpec(bz.shape, lambda i: (0,)),
            pl.BlockSpec(bh.shape, lambda i: (0,)),
        ],
        out_specs=pl.BlockSpec((BLK, d), lambda i: (i, 0)),
        out_shape=jax.ShapeDtypeStruct((n_pad, d), jnp.float32),
    )(xp, s_part, dinv, dinv2, Wx_z, Wx_h, W_gcn, bz, bh)

    s2_part = segsum(y2, src64, dst64, zrow)

    wl_row = W_lin.reshape(1, -1)
    outp = pl.pallas_call(
        _final_body,
        grid=(grid,),
        in_specs=[
            pl.BlockSpec((NC, BLK, d), lambda i: (0, i, 0)),
            pl.BlockSpec((BLK, d), lambda i: (i, 0)),
            pl.BlockSpec((BLK, 1), lambda i: (i, 0)),
            pl.BlockSpec(b_gcn.shape, lambda i: (0,)),
            pl.BlockSpec((1, d), lambda i: (0, 0)),
            pl.BlockSpec(b_lin.shape, lambda i: (0,)),
        ],
        out_specs=pl.BlockSpec((BLK, 1), lambda i: (i, 0)),
        out_shape=jax.ShapeDtypeStruct((n_pad, 1), jnp.float32),
    )(s2_part, y2, dinv2, b_gcn, wl_row, b_lin)

    return outp[:n, 0]
